# baseline (speedup 1.0000x reference)
import functools

import jax
import jax.numpy as jnp
from jax import lax
from jax.experimental import pallas as pl
from jax.experimental.pallas import tpu as pltpu
from jax.experimental.pallas import tpu_sc as plsc

B = 4096
CTX = 20
EMBED = 64
VOCAB = 1000000
NPAIR = VOCAB // 2
NC = 2
NS = 16
NW = NC * NS
BPW = B // NW
QB = 16
NQ = BPW // QB
NSLAB = VOCAB // 128
FULL_PER_TILE = NSLAB // NW
EXTRA_TILES = NSLAB - FULL_PER_TILE * NW


def _sc_reformat(wt, w2_tail):
    mesh = plsc.VectorSubcoreMesh(core_axis_name="c", subcore_axis_name="s")

    @functools.partial(
        pl.kernel,
        mesh=mesh,
        compiler_params=pltpu.CompilerParams(needs_layout_passes=False),
        out_type=jax.ShapeDtypeStruct((NPAIR, 128), jnp.float32),
        scratch_types=[
            pltpu.VMEM((2, EMBED, 128), jnp.float32),
            pltpu.VMEM((2, EMBED, 128), jnp.float32),
            pltpu.SemaphoreType.DMA,
            pltpu.SemaphoreType.DMA,
        ],
    )
    def k(wt_hbm, tail_hbm, w2_out, in2, out2, isem, osem):
        wid = lax.axis_index("s") * NC + lax.axis_index("c")
        nfull = FULL_PER_TILE + jnp.where(wid < EXTRA_TILES, 1, 0)

        def slab_of(g):
            return wid + NW * g

        def in_lane(g):
            return pl.multiple_of(slab_of(g) * 128, 128)

        for par in range(2):
            pltpu.async_copy(wt_hbm.at[:, pl.ds(in_lane(par), 128)],
                             in2.at[par], isem)

        def transpose_slab(par, src_cols, dst_rows, dst_base):
            iotas = [jax.lax.iota(jnp.int32, 16) + j * 16
                     for j in range(EMBED // 16)]
            for w in range(2 * dst_rows):
                wv = jnp.full((16,), w, jnp.int32)
                r, half = w // 2, w % 2
                for j in range(EMBED // 16):
                    col = plsc.load_gather(in2.at[par], [iotas[j], wv])
                    out2[par, r, pl.ds(half * 64 + j * 16, 16)] = col
            pltpu.async_copy(out2.at[par],
                             w2_out.at[pl.ds(dst_base, 64)], osem)

        def gbody(g2, carry):
            for par in range(2):
                g = g2 * 2 + par
                live = g < nfull

                @pl.when(live)
                def _():
                    pltpu.make_async_copy(
                        wt_hbm.at[:, pl.ds(0, 128)], in2.at[par], isem
                    ).wait()

                @pl.when(live & (g >= 2))
                def _():
                    pltpu.make_async_copy(
                        out2.at[par], w2_out.at[pl.ds(0, 64)], osem
                    ).wait()

                @pl.when(live)
                def _():
                    dst = pl.multiple_of(slab_of(g) * 64, 64)
                    transpose_slab(par, 128, 64, dst)

                @pl.when((g + 2) < nfull)
                def _():
                    pltpu.async_copy(
                        wt_hbm.at[:, pl.ds(in_lane(g + 2), 128)],
                        in2.at[par], isem)
            return carry

        lax.fori_loop(0, (FULL_PER_TILE + 3) // 2, gbody, 0)

        for par in range(2):
            @pl.when(jnp.where(wid < EXTRA_TILES, FULL_PER_TILE + 1,
                               FULL_PER_TILE) > par)
            def _():
                pltpu.make_async_copy(
                    out2.at[par], w2_out.at[pl.ds(0, 64)], osem).wait()

        @pl.when(wid == NW - 1)
        def _():
            pltpu.sync_copy(tail_hbm, out2.at[0].at[pl.ds(0, 32), :])
            pltpu.sync_copy(out2.at[0].at[pl.ds(0, 32), :],
                            w2_out.at[pl.ds(NSLAB * 64, 32)])

    return k(wt, w2_tail)


def _sc_lookup(pair_t, par_t, o_pair, o_par, w2):
    mesh = plsc.VectorSubcoreMesh(core_axis_name="c", subcore_axis_name="s")

    @functools.partial(
        pl.kernel,
        mesh=mesh,
        out_type=(
            jax.ShapeDtypeStruct((B, EMBED), jnp.float32),
            jax.ShapeDtypeStruct((B, EMBED), jnp.float32),
        ),
        scratch_types=[
            pltpu.VMEM((CTX, BPW), jnp.int32),
            pltpu.VMEM((CTX, BPW), jnp.int32),
            pltpu.VMEM((BPW,), jnp.int32),
            pltpu.VMEM((BPW,), jnp.int32),
            pltpu.VMEM((CTX, QB, 128), jnp.float32),
            pltpu.VMEM((BPW, 128), jnp.float32),
            pltpu.VMEM((BPW, EMBED), jnp.float32),
            pltpu.VMEM((BPW, EMBED), jnp.float32),
            pltpu.SemaphoreType.DMA,
        ],
    )
    def k(pair_hbm, par_hbm, opair_hbm, opar_hbm, w2_hbm, in_out, out_out,
          idx_v, par_v, oidx_v, opar_v, bufs, obuf, acc_v, orow_v, sem):
        wid = lax.axis_index("s") * NC + lax.axis_index("c")
        base = wid * BPW

        pltpu.sync_copy(pair_hbm.at[:, pl.ds(base, BPW)], idx_v)
        pltpu.sync_copy(par_hbm.at[:, pl.ds(base, BPW)], par_v)
        pltpu.sync_copy(opair_hbm.at[pl.ds(base, BPW)], oidx_v)
        pltpu.sync_copy(opar_hbm.at[pl.ds(base, BPW)], opar_v)

        pltpu.async_copy(w2_hbm.at[oidx_v], obuf, sem).wait()

        def obody(g, carry):
            vecp = opar_v[pl.ds(g * 16, 16)]
            for kk in range(16):
                p = vecp[kk]
                for j in range(EMBED // 16):
                    orow_v[g * 16 + kk, pl.ds(j * 16, 16)] = (
                        obuf[g * 16 + kk, pl.ds(p + j * 16, 16)])
            return carry

        lax.fori_loop(0, BPW // 16, obody, 0)
        pltpu.sync_copy(orow_v, out_out.at[pl.ds(base, BPW)])

        for q in range(NQ):
            cps = [
                pltpu.async_copy(
                    w2_hbm.at[idx_v.at[c, pl.ds(q * QB, QB)]],
                    bufs.at[c], sem)
                for c in range(CTX)
            ]
            for cp in cps:
                cp.wait()

            for j in range(EMBED // 16):

                def acc_body(c, carry, q=q, j=j):
                    vecp = par_v[c, pl.ds(q * QB, 16)]
                    out = []
                    for b in range(QB):
                        p = vecp[b]
                        out.append(carry[b]
                                   + bufs[c, b, pl.ds(p + j * 16, 16)])
                    return tuple(out)

                init = acc_body(0, (jnp.zeros((16,), jnp.float32),) * QB)
                res = lax.fori_loop(1, CTX, acc_body, init)
                for b in range(QB):
                    acc_v[q * QB + b, pl.ds(j * 16, 16)] = res[b]

        pltpu.sync_copy(acc_v, in_out.at[pl.ds(base, BPW)])

    return k(pair_t, par_t, o_pair, o_par, w2)


def _tc_score(in_e, out_e):
    m_blk = 512

    def body(a_ref, b_ref, o_ref):
        a = a_ref[...].astype(jnp.bfloat16)
        bt = b_ref[...].astype(jnp.bfloat16)
        s = lax.dot_general(a, bt, (((1,), (1,)), ((), ())),
                            preferred_element_type=jnp.float32)
        o_ref[...] = jnp.minimum(s, 0.0) - jnp.log1p(jnp.exp(-jnp.abs(s)))

    return pl.pallas_call(
        body,
        grid=(B // m_blk,),
        in_specs=[
            pl.BlockSpec((m_blk, EMBED), lambda m: (m, 0)),
            pl.BlockSpec((B, EMBED), lambda m: (0, 0)),
        ],
        out_specs=pl.BlockSpec((m_blk, B), lambda m: (m, 0)),
        out_shape=jax.ShapeDtypeStruct((B, B), jnp.float32),
    )(in_e, out_e)


def kernel(i, o, W):
    i32 = i.astype(jnp.int32)
    o32 = o.astype(jnp.int32)
    w2_tail = W[NSLAB * 128:, :].reshape(32, 128)
    w2 = _sc_reformat(W.T, w2_tail)
    pair_t = (i32 >> 1).T
    par_t = ((i32 & 1) * 64).T
    in_e, out_e = _sc_lookup(pair_t, par_t, o32 >> 1, (o32 & 1) * 64, w2)
    return _tc_score(in_e, out_e)


# speedup vs baseline: 5.2503x; 5.2503x over previous
import functools

import jax
import jax.numpy as jnp
from jax import lax
from jax.experimental import pallas as pl
from jax.experimental.pallas import tpu as pltpu
from jax.experimental.pallas import tpu_sc as plsc

B = 4096
CTX = 20
EMBED = 64
NC = 2
NS = 16
NW = NC * NS
BPW = B // NW
QB = 16
NQ = BPW // QB


def _sc_gather(i_t, o_arr, w):
    mesh = plsc.VectorSubcoreMesh(core_axis_name="c", subcore_axis_name="s")

    @functools.partial(
        pl.kernel,
        mesh=mesh,
        out_type=(
            jax.ShapeDtypeStruct((B, EMBED), jnp.float32),
            jax.ShapeDtypeStruct((B, EMBED), jnp.float32),
        ),
        scratch_types=[
            pltpu.VMEM((CTX, BPW), jnp.int32),
            pltpu.VMEM((BPW,), jnp.int32),
            pltpu.VMEM((CTX * QB, EMBED), jnp.float32),
            pltpu.VMEM((BPW, EMBED), jnp.float32),
            pltpu.VMEM((BPW, EMBED), jnp.float32),
            pltpu.SemaphoreType.DMA,
            pltpu.SemaphoreType.DMA,
        ],
    )
    def k(i_hbm, o_hbm, w_hbm, in_out, out_out,
          idx_v, oidx_v, bufs, acc_v, orow_v, osem, sem):
        wid = lax.axis_index("s") * NC + lax.axis_index("c")
        base = wid * BPW

        pltpu.sync_copy(i_hbm.at[:, pl.ds(base, BPW)], idx_v)
        pltpu.sync_copy(o_hbm.at[pl.ds(base, BPW)], oidx_v)

        def ofire(g, carry):
            vec = oidx_v[pl.ds(g * 16, 16)]
            for kk in range(16):
                pltpu.async_copy(w_hbm.at[pl.ds(vec[kk], 1), :],
                                 orow_v.at[pl.ds(g * 16 + kk, 1), :], osem)
            return carry

        lax.fori_loop(0, BPW // 16, ofire, 0)
        pltpu.make_async_copy(w_hbm.at[pl.ds(0, BPW), :], orow_v, osem).wait()
        pltpu.sync_copy(orow_v, out_out.at[pl.ds(base, BPW)])

        for q in range(NQ):

            def fire(c, carry, q=q):
                vec = idx_v[c, pl.ds(q * QB, 16)]
                for kk in range(16):
                    pltpu.async_copy(
                        w_hbm.at[pl.ds(vec[kk], 1), :],
                        bufs.at[pl.ds(c * QB + kk, 1), :], sem)
                return carry

            lax.fori_loop(0, CTX, fire, 0)
            pltpu.make_async_copy(w_hbm.at[pl.ds(0, CTX * QB), :], bufs,
                                  sem).wait()

            def body(b, carry, q=q):
                for j in range(EMBED // 16):
                    a = bufs[b, pl.ds(j * 16, 16)]
                    for c in range(1, CTX):
                        a = a + bufs[c * QB + b, pl.ds(j * 16, 16)]
                    acc_v[q * QB + b, pl.ds(j * 16, 16)] = a
                return carry

            lax.fori_loop(0, QB, body, 0)

        pltpu.sync_copy(acc_v, in_out.at[pl.ds(base, BPW)])

    return k(i_t, o_arr, w)


def _tc_score(in_e, out_e):
    m_blk = 512

    def body(a_ref, b_ref, o_ref):
        a = a_ref[...].astype(jnp.bfloat16)
        bt = b_ref[...].astype(jnp.bfloat16)
        s = lax.dot_general(a, bt, (((1,), (1,)), ((), ())),
                            preferred_element_type=jnp.float32)
        o_ref[...] = jnp.minimum(s, 0.0) - jnp.log1p(jnp.exp(-jnp.abs(s)))

    return pl.pallas_call(
        body,
        grid=(B // m_blk,),
        in_specs=[
            pl.BlockSpec((m_blk, EMBED), lambda m: (m, 0)),
            pl.BlockSpec((B, EMBED), lambda m: (0, 0)),
        ],
        out_specs=pl.BlockSpec((m_blk, B), lambda m: (m, 0)),
        out_shape=jax.ShapeDtypeStruct((B, B), jnp.float32),
    )(in_e, out_e)


def kernel(i, o, W):
    i32 = i.astype(jnp.int32)
    o32 = o.astype(jnp.int32)
    w_rm = jax.lax.optimization_barrier(W.T).T
    in_e, out_e = _sc_gather(i32.T, o32, w_rm)
    return _tc_score(in_e, out_e)


# speedup vs baseline: 5.3952x; 1.0276x over previous
import functools

import jax
import jax.numpy as jnp
from jax import lax
from jax.experimental import pallas as pl
from jax.experimental.pallas import tpu as pltpu
from jax.experimental.pallas import tpu_sc as plsc

B = 4096
CTX = 20
EMBED = 64
NC = 2
NS = 16
NW = NC * NS
BPW = B // NW
QB = 16
NQ = BPW // QB


def _sc_gather(i_t, o_arr, w):
    mesh = plsc.VectorSubcoreMesh(core_axis_name="c", subcore_axis_name="s")

    @functools.partial(
        pl.kernel,
        mesh=mesh,
        out_type=(
            jax.ShapeDtypeStruct((B, EMBED), jnp.float32),
            jax.ShapeDtypeStruct((B, EMBED), jnp.float32),
        ),
        scratch_types=[
            pltpu.VMEM((CTX, BPW), jnp.int32),
            pltpu.VMEM((BPW,), jnp.int32),
            pltpu.VMEM((2, CTX * QB, EMBED), jnp.float32),
            pltpu.VMEM((BPW, EMBED), jnp.float32),
            pltpu.VMEM((BPW, EMBED), jnp.float32),
            pltpu.SemaphoreType.DMA,
            pltpu.SemaphoreType.DMA,
            pltpu.SemaphoreType.DMA,
        ],
    )
    def k(i_hbm, o_hbm, w_hbm, in_out, out_out,
          idx_v, oidx_v, bufs, acc_v, orow_v, osem, sem0, sem1):
        wid = lax.axis_index("s") * NC + lax.axis_index("c")
        base = wid * BPW

        pltpu.sync_copy(i_hbm.at[:, pl.ds(base, BPW)], idx_v)
        pltpu.sync_copy(o_hbm.at[pl.ds(base, BPW)], oidx_v)

        def ofire(g, carry):
            vec = oidx_v[pl.ds(g * 16, 16)]
            for kk in range(16):
                pltpu.async_copy(w_hbm.at[pl.ds(vec[kk], 1), :],
                                 orow_v.at[pl.ds(g * 16 + kk, 1), :], osem)
            return carry

        lax.fori_loop(0, BPW // 16, ofire, 0)
        pltpu.make_async_copy(w_hbm.at[pl.ds(0, BPW), :], orow_v, osem).wait()
        pltpu.sync_copy(orow_v, out_out.at[pl.ds(base, BPW)])

        sems = (sem0, sem1)

        def fire(q):
            def fire_c(c, carry):
                vec = idx_v[c, pl.ds(q * QB, 16)]
                for kk in range(16):
                    pltpu.async_copy(
                        w_hbm.at[pl.ds(vec[kk], 1), :],
                        bufs.at[q % 2].at[pl.ds(c * QB + kk, 1), :],
                        sems[q % 2])
                return carry

            lax.fori_loop(0, CTX, fire_c, 0)

        fire(0)
        for q in range(NQ):
            if q + 1 < NQ:
                fire(q + 1)
            pltpu.make_async_copy(w_hbm.at[pl.ds(0, CTX * QB), :],
                                  bufs.at[q % 2], sems[q % 2]).wait()

            def body(b, carry, q=q):
                for j in range(EMBED // 16):
                    a = bufs[q % 2, b, pl.ds(j * 16, 16)]
                    for c in range(1, CTX):
                        a = a + bufs[q % 2, c * QB + b, pl.ds(j * 16, 16)]
                    acc_v[q * QB + b, pl.ds(j * 16, 16)] = a
                return carry

            lax.fori_loop(0, QB, body, 0)

        pltpu.sync_copy(acc_v, in_out.at[pl.ds(base, BPW)])

    return k(i_t, o_arr, w)


def _tc_score(in_e, out_e):
    m_blk = 512

    def body(a_ref, b_ref, o_ref):
        a = a_ref[...].astype(jnp.bfloat16)
        bt = b_ref[...].astype(jnp.bfloat16)
        s = lax.dot_general(a, bt, (((1,), (1,)), ((), ())),
                            preferred_element_type=jnp.float32)
        o_ref[...] = jnp.minimum(s, 0.0) - jnp.log1p(jnp.exp(-jnp.abs(s)))

    return pl.pallas_call(
        body,
        grid=(B // m_blk,),
        in_specs=[
            pl.BlockSpec((m_blk, EMBED), lambda m: (m, 0)),
            pl.BlockSpec((B, EMBED), lambda m: (0, 0)),
        ],
        out_specs=pl.BlockSpec((m_blk, B), lambda m: (m, 0)),
        out_shape=jax.ShapeDtypeStruct((B, B), jnp.float32),
    )(in_e, out_e)


def kernel(i, o, W):
    i32 = i.astype(jnp.int32)
    o32 = o.astype(jnp.int32)
    w_rm = jax.lax.optimization_barrier(W.T).T
    in_e, out_e = _sc_gather(i32.T, o32, w_rm)
    return _tc_score(in_e, out_e)
